# baseline (speedup 1.0000x reference)
import functools

import jax
import jax.numpy as jnp
from jax import lax
from jax.experimental import pallas as pl
from jax.experimental.pallas import tpu as pltpu
from jax.experimental.pallas import tpu_sc as plsc

BS = 64
NQ = 1024
L = 16
NC = 2
NS = 16
NW = NC * NS
ROWS_PER_W = BS // NW
CHUNKS = NQ // L


def _matcher_body(coords_hbm, params_hbm, idx_hbm, cnt_hbm, delta_hbm,
                  crow, prow, irow, drow, cvec):
    wid = lax.axis_index("s") * NC + lax.axis_index("c")
    iota = lax.iota(jnp.int32, L)
    nq_v = jnp.full((L,), NQ, jnp.int32)
    one_v = jnp.full((L,), 1, jnp.int32)
    zero_v = jnp.zeros((L,), jnp.int32)
    half_v = jnp.full((L,), 0.5, jnp.float32)

    for rr in range(ROWS_PER_W):
        r = wid * ROWS_PER_W + rr
        pltpu.sync_copy(coords_hbm.at[r], crow)
        pltpu.sync_copy(params_hbm.at[r], prow)

        cx_v = prow[pl.ds(0 * L, L)]
        cy_v = prow[pl.ds(1 * L, L)]
        w_v = prow[pl.ds(2 * L, L)]
        h_v = prow[pl.ds(3 * L, L)]
        xmin_v = cx_v - w_v * half_v
        xmax_v = cx_v + w_v * half_v
        ymin_v = cy_v - h_v * half_v
        ymax_v = cy_v + h_v * half_v

        def chunk(j, c):
            qi = j * L + iota
            ex = qi * 2
            ey = ex + 1
            xv = plsc.load_gather(crow, [ex])
            yv = plsc.load_gather(crow, [ey])
            ind_x = (xv - xmin_v) * (xv - xmax_v)
            ind_y = (yv - ymin_v) * (yv - ymax_v)
            m = (ind_x < 0.0) & (ind_y < 0.0)
            plsc.store_scatter(drow, [ex], cx_v - xv)
            plsc.store_scatter(drow, [ey], cy_v - yv)
            irow[pl.ds(j * L, L)] = nq_v
            prefix = plsc.cumsum(jnp.where(m, one_v, zero_v))
            pos = c + prefix - one_v
            plsc.store_scatter(irow, [pos], qi, mask=m)
            return c + plsc.all_reduce_population_count(m)

        c_fin = lax.fori_loop(0, CHUNKS, chunk, zero_v)
        cvec[...] = c_fin
        pltpu.sync_copy(irow, idx_hbm.at[r])
        pltpu.sync_copy(cvec, cnt_hbm.at[r])
        pltpu.sync_copy(drow, delta_hbm.at[r])


_matcher = functools.partial(
    pl.kernel,
    mesh=plsc.VectorSubcoreMesh(core_axis_name="c", subcore_axis_name="s"),
    compiler_params=pltpu.CompilerParams(needs_layout_passes=False),
    out_type=(
        jax.ShapeDtypeStruct((BS, NQ), jnp.int32),
        jax.ShapeDtypeStruct((BS, L), jnp.int32),
        jax.ShapeDtypeStruct((BS, 2 * NQ), jnp.float32),
    ),
    scratch_types=[
        pltpu.VMEM((2 * NQ,), jnp.float32),
        pltpu.VMEM((4 * L,), jnp.float32),
        pltpu.VMEM((NQ,), jnp.int32),
        pltpu.VMEM((2 * NQ,), jnp.float32),
        pltpu.VMEM((L,), jnp.int32),
    ],
)(_matcher_body)


def kernel(bilinear_coords, boxes):
    coords = bilinear_coords.reshape(BS, 2 * NQ)
    params = jnp.broadcast_to(
        boxes.reshape(BS, 4, 1), (BS, 4, L)).reshape(BS, 4 * L)
    idx, cnt, delta = _matcher(coords, params)
    padded_idx = idx.astype(jnp.int64)
    counts = cnt[:, 0].astype(jnp.int64)
    return padded_idx, counts, delta.reshape(BS, NQ, 2)


# speedup vs baseline: 1.1170x; 1.1170x over previous
import functools

import jax
import jax.numpy as jnp
from jax import lax
from jax.experimental import pallas as pl
from jax.experimental.pallas import tpu as pltpu
from jax.experimental.pallas import tpu_sc as plsc

BS = 64
NQ = 1024
L = 16
NC = 2
NS = 16
CHUNKS = NQ // L


def _matcher_body(coords_hbm, boxes_hbm, idx_hbm, cnt_hbm, delta_hbm,
                  boxv, crowA, crowB, irowA, irowB, drowA, drowB,
                  sbuf, outv, cvecv, shared, semA, semB, semX, semO):
    cid = lax.axis_index("c")
    sid = lax.axis_index("s")
    rA = cid * 32 + sid * 2
    rB = rA + 1
    iota = lax.iota(jnp.int32, L)
    nq_v = jnp.full((L,), NQ, jnp.int32)
    one_v = jnp.full((L,), 1, jnp.int32)
    zero_v = jnp.zeros((L,), jnp.int32)
    half_v = jnp.full((L,), 0.5, jnp.float32)

    hA = pltpu.async_copy(coords_hbm.at[rA], crowA, semA)
    hB = pltpu.async_copy(coords_hbm.at[rB], crowB, semB)
    hX = pltpu.async_copy(boxes_hbm, boxv, semX)

    def fill(j, _):
        irowA[pl.ds(j * L, L)] = nq_v
        irowB[pl.ds(j * L, L)] = nq_v
        return 0

    lax.fori_loop(0, CHUNKS, fill, 0)
    hX.wait()

    def box_vecs(r):
        base = zero_v + r * 4
        cx = plsc.load_gather(boxv, [base])
        cy = plsc.load_gather(boxv, [base + 1])
        w = plsc.load_gather(boxv, [base + 2])
        h = plsc.load_gather(boxv, [base + 3])
        return (cx, cy, cx - w * half_v, cx + w * half_v,
                cy - h * half_v, cy + h * half_v)

    cxA, cyA, xminA, xmaxA, yminA, ymaxA = box_vecs(rA)
    cxB, cyB, xminB, xmaxB, yminB, ymaxB = box_vecs(rB)
    hA.wait()
    hB.wait()

    def chunk(j, carry):
        cA, cB = carry
        qi = j * L + iota
        ex = qi * 2
        ey = ex + one_v
        xA = plsc.load_gather(crowA, [ex])
        yA = plsc.load_gather(crowA, [ey])
        xB = plsc.load_gather(crowB, [ex])
        yB = plsc.load_gather(crowB, [ey])
        mA = (((xA - xminA) * (xA - xmaxA) < 0.0)
              & ((yA - yminA) * (yA - ymaxA) < 0.0))
        mB = (((xB - xminB) * (xB - xmaxB) < 0.0)
              & ((yB - yminB) * (yB - ymaxB) < 0.0))
        plsc.store_scatter(drowA, [ex], cxA - xA)
        plsc.store_scatter(drowA, [ey], cyA - yA)
        plsc.store_scatter(drowB, [ex], cxB - xB)
        plsc.store_scatter(drowB, [ey], cyB - yB)
        pA = plsc.cumsum(jnp.where(mA, one_v, zero_v))
        pB = plsc.cumsum(jnp.where(mB, one_v, zero_v))
        plsc.store_scatter(irowA, [cA + pA - one_v], qi, mask=mA)
        plsc.store_scatter(irowB, [cB + pB - one_v], qi, mask=mB)
        return (cA + plsc.all_reduce_population_count(mA),
                cB + plsc.all_reduce_population_count(mB))

    cA_fin, cB_fin = lax.fori_loop(0, CHUNKS, chunk, (zero_v, zero_v))

    hoA = pltpu.async_copy(irowA, idx_hbm.at[rA], semO)
    hoB = pltpu.async_copy(irowB, idx_hbm.at[rB], semO)
    hdA = pltpu.async_copy(drowA, delta_hbm.at[rA], semO)
    hdB = pltpu.async_copy(drowB, delta_hbm.at[rB], semO)

    cvecv[...] = jnp.where(iota == 0, cA_fin, cB_fin)
    pltpu.sync_copy(cvecv, shared.at[pl.ds(sid * L, L)])
    plsc.subcore_barrier()

    @pl.when(sid == 0)
    def _assemble():
        pltpu.sync_copy(shared, sbuf)
        for t in range(2):
            iv = t * L + iota
            flat = lax.shift_right_logical(iv, 1) * L + lax.bitwise_and(iv, one_v)
            outv[pl.ds(t * L, L)] = plsc.load_gather(sbuf, [flat])
        pltpu.sync_copy(outv, cnt_hbm.at[pl.ds(cid * 32, 32)])

    hoA.wait()
    hoB.wait()
    hdA.wait()
    hdB.wait()


_matcher = functools.partial(
    pl.kernel,
    mesh=plsc.VectorSubcoreMesh(core_axis_name="c", subcore_axis_name="s"),
    compiler_params=pltpu.CompilerParams(needs_layout_passes=False),
    out_type=(
        jax.ShapeDtypeStruct((BS, NQ), jnp.int32),
        jax.ShapeDtypeStruct((BS,), jnp.int32),
        jax.ShapeDtypeStruct((BS, 2 * NQ), jnp.float32),
    ),
    scratch_types=[
        pltpu.VMEM((4 * BS,), jnp.float32),
        pltpu.VMEM((2 * NQ,), jnp.float32),
        pltpu.VMEM((2 * NQ,), jnp.float32),
        pltpu.VMEM((NQ,), jnp.int32),
        pltpu.VMEM((NQ,), jnp.int32),
        pltpu.VMEM((2 * NQ,), jnp.float32),
        pltpu.VMEM((2 * NQ,), jnp.float32),
        pltpu.VMEM((NS * L,), jnp.int32),
        pltpu.VMEM((2 * NS,), jnp.int32),
        pltpu.VMEM((L,), jnp.int32),
        pltpu.VMEM_SHARED((NS * L,), jnp.int32),
        pltpu.SemaphoreType.DMA,
        pltpu.SemaphoreType.DMA,
        pltpu.SemaphoreType.DMA,
        pltpu.SemaphoreType.DMA,
    ],
)(_matcher_body)


def kernel(bilinear_coords, boxes):
    coords = bilinear_coords.reshape(BS, 2 * NQ)
    idx, cnt, delta = _matcher(coords, boxes.reshape(4 * BS))
    return (idx.astype(jnp.int64), cnt.astype(jnp.int64),
            delta.reshape(BS, NQ, 2))


# speedup vs baseline: 1.2960x; 1.1602x over previous
import functools

import jax
import jax.numpy as jnp
from jax import lax
from jax.experimental import pallas as pl
from jax.experimental.pallas import tpu as pltpu
from jax.experimental.pallas import tpu_sc as plsc

BS = 64
NQ = 1024
L = 16
NC = 2
NS = 16
CHUNKS = NQ // L


def _matcher_body(coords_hbm, boxes_hbm, idx_hbm, cnt_hbm, delta_hbm,
                  boxv, crowA, crowB, irowA, irowB, drowA, drowB,
                  sbuf, outv, cvecv, shared, semA, semB, semX, semO):
    cid = lax.axis_index("c")
    sid = lax.axis_index("s")
    rA = cid * 32 + sid * 2
    rB = rA + 1
    iota = lax.iota(jnp.int32, L)
    nq_v = jnp.full((L,), NQ, jnp.int32)
    one_v = jnp.full((L,), 1, jnp.int32)
    zero_v = jnp.zeros((L,), jnp.int32)
    half_v = jnp.full((L,), 0.5, jnp.float32)
    c127_v = jnp.full((L,), 127, jnp.int32)

    hA = pltpu.async_copy(coords_hbm.at[rA], crowA, semA)
    hB = pltpu.async_copy(coords_hbm.at[rB], crowB, semB)
    hX = pltpu.async_copy(boxes_hbm, boxv, semX)

    def fill(j, _):
        row = zero_v + lax.shift_right_logical(j, 3)
        col = (j & 7) * L + iota
        plsc.store_scatter(irowA, [row, col], nq_v)
        plsc.store_scatter(irowB, [row, col], nq_v)
        return 0

    lax.fori_loop(0, CHUNKS, fill, 0)
    hX.wait()

    def box_vecs(r):
        base = zero_v + r * 4
        cx = plsc.load_gather(boxv, [base])
        cy = plsc.load_gather(boxv, [base + 1])
        w = plsc.load_gather(boxv, [base + 2])
        h = plsc.load_gather(boxv, [base + 3])
        return (cx, cy, cx - w * half_v, cx + w * half_v,
                cy - h * half_v, cy + h * half_v)

    cxA, cyA, xminA, xmaxA, yminA, ymaxA = box_vecs(rA)
    cxB, cyB, xminB, xmaxB, yminB, ymaxB = box_vecs(rB)
    hA.wait()
    hB.wait()

    def chunk(j, carry):
        cA, cB = carry
        qi = j * L + iota
        srow = zero_v + lax.shift_right_logical(j, 3) * 2
        scol = (j & 7) * L + iota
        xA = plsc.load_gather(crowA, [srow, scol])
        yA = plsc.load_gather(crowA, [srow + 1, scol])
        xB = plsc.load_gather(crowB, [srow, scol])
        yB = plsc.load_gather(crowB, [srow + 1, scol])
        mA = (((xA - xminA) * (xA - xmaxA) < 0.0)
              & ((yA - yminA) * (yA - ymaxA) < 0.0))
        mB = (((xB - xminB) * (xB - xmaxB) < 0.0)
              & ((yB - yminB) * (yB - ymaxB) < 0.0))
        plsc.store_scatter(drowA, [srow, scol], cxA - xA)
        plsc.store_scatter(drowA, [srow + 1, scol], cyA - yA)
        plsc.store_scatter(drowB, [srow, scol], cxB - xB)
        plsc.store_scatter(drowB, [srow + 1, scol], cyB - yB)
        pA = plsc.cumsum(jnp.where(mA, one_v, zero_v))
        pB = plsc.cumsum(jnp.where(mB, one_v, zero_v))
        posA = cA + pA - one_v
        posB = cB + pB - one_v
        plsc.store_scatter(
            irowA, [lax.shift_right_logical(posA, 7), posA & c127_v],
            qi, mask=mA)
        plsc.store_scatter(
            irowB, [lax.shift_right_logical(posB, 7), posB & c127_v],
            qi, mask=mB)
        return (cA + plsc.all_reduce_population_count(mA),
                cB + plsc.all_reduce_population_count(mB))

    cA_fin, cB_fin = lax.fori_loop(0, CHUNKS, chunk, (zero_v, zero_v))

    hoA = pltpu.async_copy(irowA, idx_hbm.at[lax.shift_right_logical(rA, 3),
                                             :, rA & 7], semO)
    hoB = pltpu.async_copy(irowB, idx_hbm.at[lax.shift_right_logical(rB, 3),
                                             :, rB & 7], semO)
    hdA = pltpu.async_copy(drowA, delta_hbm.at[rA], semO)
    hdB = pltpu.async_copy(drowB, delta_hbm.at[rB], semO)

    cvecv[...] = jnp.where(iota == 0, cA_fin, cB_fin)
    pltpu.sync_copy(cvecv, shared.at[pl.ds(sid * L, L)])
    plsc.subcore_barrier()

    @pl.when(sid == 0)
    def _assemble():
        pltpu.sync_copy(shared, sbuf)
        for t in range(2):
            iv = t * L + iota
            flat = lax.shift_right_logical(iv, 1) * L + lax.bitwise_and(iv, one_v)
            outv[pl.ds(t * L, L)] = plsc.load_gather(sbuf, [flat])
        pltpu.sync_copy(outv, cnt_hbm.at[pl.ds(cid * 32, 32)])

    hoA.wait()
    hoB.wait()
    hdA.wait()
    hdB.wait()


_matcher = functools.partial(
    pl.kernel,
    mesh=plsc.VectorSubcoreMesh(core_axis_name="c", subcore_axis_name="s"),
    compiler_params=pltpu.CompilerParams(needs_layout_passes=False),
    out_type=(
        jax.ShapeDtypeStruct((8, 8, 8, 128), jnp.int32),
        jax.ShapeDtypeStruct((BS,), jnp.int32),
        jax.ShapeDtypeStruct((BS, 16, 128), jnp.float32),
    ),
    scratch_types=[
        pltpu.VMEM((4 * BS,), jnp.float32),
        pltpu.VMEM((16, 128), jnp.float32),
        pltpu.VMEM((16, 128), jnp.float32),
        pltpu.VMEM((8, 128), jnp.int32),
        pltpu.VMEM((8, 128), jnp.int32),
        pltpu.VMEM((16, 128), jnp.float32),
        pltpu.VMEM((16, 128), jnp.float32),
        pltpu.VMEM((NS * L,), jnp.int32),
        pltpu.VMEM((2 * NS,), jnp.int32),
        pltpu.VMEM((L,), jnp.int32),
        pltpu.VMEM_SHARED((NS * L,), jnp.int32),
        pltpu.SemaphoreType.DMA,
        pltpu.SemaphoreType.DMA,
        pltpu.SemaphoreType.DMA,
        pltpu.SemaphoreType.DMA,
    ],
)(_matcher_body)


def kernel(bilinear_coords, boxes):
    coords = (bilinear_coords.reshape(BS, 8, 128, 2)
              .transpose(0, 1, 3, 2).reshape(BS, 16, 128))
    idx4, cnt, delta = _matcher(coords, boxes.reshape(4 * BS))
    padded_idx = idx4.transpose(0, 2, 1, 3).reshape(BS, NQ)
    delta_out = (delta.reshape(BS, 8, 2, 128)
                 .transpose(0, 1, 3, 2).reshape(BS, NQ, 2))
    return (padded_idx.astype(jnp.int64), cnt.astype(jnp.int64), delta_out)
